# baseline (speedup 1.0000x reference)
import functools

import jax
import jax.numpy as jnp
from jax import lax
from jax.experimental import pallas as pl
from jax.experimental.pallas import tpu as pltpu
from jax.experimental.pallas import tpu_sc as plsc

PHI_FLOAT = 1.618033988749895
EMBED_DIM = 256
LANES = 16
CHUNK = 128


def _make_sc_gather(B, V):
    info = plsc.get_sparse_core_info()
    num_workers = info.num_cores * info.num_subcores
    b_per_w = B // num_workers
    n_chunks = b_per_w // CHUNK
    mesh = plsc.VectorSubcoreMesh(core_axis_name="c", subcore_axis_name="s")

    @functools.partial(
        pl.kernel,
        mesh=mesh,
        out_type=jax.ShapeDtypeStruct((B, EMBED_DIM), jnp.float32),
        scratch_types=[
            pltpu.VMEM((b_per_w,), jnp.int32),
            pltpu.VMEM((CHUNK, EMBED_DIM), jnp.float32),
            pltpu.SemaphoreType.DMA,
        ],
    )
    def gather_kernel(table_hbm, idx_hbm, out_hbm, idx_v, rows_v, sem):
        wid = lax.axis_index("s") * info.num_cores + lax.axis_index("c")
        base = wid * b_per_w
        pltpu.sync_copy(idx_hbm.at[pl.ds(base, b_per_w)], idx_v)

        def chunk_body(g, _):
            idx_slice = idx_v.at[pl.ds(g * CHUNK, CHUNK)]
            pltpu.async_copy(table_hbm.at[idx_slice], rows_v, sem).wait()

            def row_body(r, _):
                for j in range(EMBED_DIM // LANES):
                    sl = pl.ds(j * LANES, LANES)
                    rows_v[r, sl] = rows_v[r, sl] * PHI_FLOAT
                return 0

            lax.fori_loop(0, CHUNK, row_body, 0)
            pltpu.sync_copy(rows_v, out_hbm.at[pl.ds(base + g * CHUNK, CHUNK)])
            return 0

        lax.fori_loop(0, n_chunks, chunk_body, 0)

    return gather_kernel


def kernel(input_ids, embedding_weight):
    batch, seq = input_ids.shape
    V, D = embedding_weight.shape
    flat_idx = input_ids.reshape(-1).astype(jnp.int32)
    B = flat_idx.shape[0]
    out = _make_sc_gather(B, V)(embedding_weight, flat_idx)
    return out.reshape(batch, seq, D)


# speedup vs baseline: 1.1681x; 1.1681x over previous
import functools

import jax
import jax.numpy as jnp
from jax import lax
from jax.experimental import pallas as pl
from jax.experimental.pallas import tpu as pltpu
from jax.experimental.pallas import tpu_sc as plsc

PHI_FLOAT = 1.618033988749895
EMBED_DIM = 256
LANES = 16
CHUNK = 64
NBUF = 4


def _make_sc_gather(B, V):
    info = plsc.get_sparse_core_info()
    num_workers = info.num_cores * info.num_subcores
    b_per_w = B // num_workers
    n_chunks = b_per_w // CHUNK
    n_outer = n_chunks // NBUF
    mesh = plsc.VectorSubcoreMesh(core_axis_name="c", subcore_axis_name="s")

    @functools.partial(
        pl.kernel,
        mesh=mesh,
        out_type=jax.ShapeDtypeStruct((B, EMBED_DIM), jnp.float32),
        scratch_types=[
            pltpu.VMEM((b_per_w,), jnp.int32),
            pltpu.VMEM((NBUF, CHUNK, EMBED_DIM), jnp.float32),
        ]
        + [pltpu.SemaphoreType.DMA] * (2 * NBUF),
    )
    def gather_kernel(table_hbm, idx_hbm, out_hbm, idx_v, rows_v, *sems):
        gsems, ssems = sems[:NBUF], sems[NBUF:]
        wid = lax.axis_index("s") * info.num_cores + lax.axis_index("c")
        base = wid * b_per_w
        pltpu.sync_copy(idx_hbm.at[pl.ds(base, b_per_w)], idx_v)

        def gather_desc(g, b):
            idx_slice = idx_v.at[pl.ds(g * CHUNK, CHUNK)]
            return pltpu.make_async_copy(
                table_hbm.at[idx_slice], rows_v.at[b], gsems[b])

        def store_desc(g, b):
            return pltpu.make_async_copy(
                rows_v.at[b], out_hbm.at[pl.ds(base + g * CHUNK, CHUNK)],
                ssems[b])

        for b in range(NBUF - 1):
            gather_desc(b, b).start()

        def outer(i, _):
            for b in range(NBUF):
                g = i * NBUF + b
                j = g + NBUF - 1
                bj = (b + NBUF - 1) % NBUF

                @pl.when(jnp.logical_and(j < n_chunks, j >= NBUF))
                def _():
                    store_desc(j - NBUF, bj).wait()
                    gather_desc(j, bj).start()

                @pl.when(jnp.logical_and(j < n_chunks, j < NBUF))
                def _():
                    gather_desc(j, bj).start()

                gather_desc(g, b).wait()

                def row_body(r, _):
                    for k in range(EMBED_DIM // LANES):
                        sl = pl.ds(k * LANES, LANES)
                        rows_v[b, r, sl] = rows_v[b, r, sl] * PHI_FLOAT
                    return 0

                lax.fori_loop(0, CHUNK, row_body, 0)
                store_desc(g, b).start()
            return 0

        lax.fori_loop(0, n_outer, outer, 0)
        for b in range(NBUF):
            store_desc(n_chunks - NBUF + b, b).wait()

    return gather_kernel


def kernel(input_ids, embedding_weight):
    batch, seq = input_ids.shape
    V, D = embedding_weight.shape
    flat_idx = input_ids.reshape(-1).astype(jnp.int32)
    B = flat_idx.shape[0]
    out = _make_sc_gather(B, V)(embedding_weight, flat_idx)
    return out.reshape(batch, seq, D)
